# baseline (device time: 44980 ns/iter reference)
import jax
import jax.numpy as jnp
from jax import lax
from jax.experimental import pallas as pl
from jax.experimental.pallas import tpu as pltpu

N_DEV = 4


def kernel(x, Win0, Wout0, Win1, Wout1, Win2, Wout2):
    m_per, d = x.shape
    M = N_DEV * m_per

    def body(x_ref, win0, wout0, win1, wout1, win2, wout2, out_ref,
             xg, acc, agc, arc, send_sems, recv_sems):
        my = lax.axis_index("i")
        left = lax.rem(my + N_DEV - 1, N_DEV)
        right = lax.rem(my + 1, N_DEV)

        barrier = pltpu.get_barrier_semaphore()
        for nbr in (left, right):
            pl.semaphore_signal(barrier, inc=1, device_id=(nbr,),
                                device_id_type=pl.DeviceIdType.MESH)
        pl.semaphore_wait(barrier, 2)

        x_bf = x_ref[:, :].astype(jnp.bfloat16)
        xg[pl.ds(my * m_per, m_per), :] = x_bf
        agc[0] = x_bf
        hop = 0
        for h in range(N_DEV - 1):
            s, r = h % 2, (h + 1) % 2
            rdma = pltpu.make_async_remote_copy(
                src_ref=agc.at[s], dst_ref=agc.at[r],
                send_sem=send_sems.at[hop], recv_sem=recv_sems.at[hop],
                device_id=(right,), device_id_type=pl.DeviceIdType.MESH)
            rdma.start()
            rdma.wait()
            origin = lax.rem(my + (N_DEV - h - 1), N_DEV)
            xg[pl.ds(origin * m_per, m_per), :] = agc[r]
            hop += 1

        for l, (win, wout) in enumerate(
                ((win0, wout0), (win1, wout1), (win2, wout2))):
            hmat = jnp.dot(xg[:, :], win[:, :].astype(jnp.bfloat16),
                           preferred_element_type=jnp.float32)
            hmat = jnp.maximum(hmat, 0.0).astype(jnp.bfloat16)
            p = jnp.dot(hmat, wout[:, :].astype(jnp.bfloat16),
                        preferred_element_type=jnp.float32)
            acc[:, :] = p
            arc[hop % 2] = p.astype(jnp.bfloat16)
            for h in range(N_DEV - 1):
                s, r = hop % 2, (hop + 1) % 2
                rdma = pltpu.make_async_remote_copy(
                    src_ref=arc.at[s], dst_ref=arc.at[r],
                    send_sem=send_sems.at[hop], recv_sem=recv_sems.at[hop],
                    device_id=(right,), device_id_type=pl.DeviceIdType.MESH)
                rdma.start()
                rdma.wait()
                acc[:, :] = acc[:, :] + arc[r].astype(jnp.float32)
                hop += 1
            if l < 2:
                xg[:, :] = acc[:, :].astype(jnp.bfloat16)
            else:
                out_ref[:, :] = acc[:, :]

    n_hops = 3 * N_DEV
    return pl.pallas_call(
        body,
        out_shape=jax.ShapeDtypeStruct((M, d), jnp.float32),
        in_specs=[pl.BlockSpec(memory_space=pltpu.VMEM)] * 7,
        out_specs=pl.BlockSpec(memory_space=pltpu.VMEM),
        scratch_shapes=[
            pltpu.VMEM((M, d), jnp.bfloat16),
            pltpu.VMEM((M, d), jnp.float32),
            pltpu.VMEM((2, m_per, d), jnp.bfloat16),
            pltpu.VMEM((2, M, d), jnp.bfloat16),
            pltpu.SemaphoreType.DMA((n_hops,)),
            pltpu.SemaphoreType.DMA((n_hops,)),
        ],
        compiler_params=pltpu.CompilerParams(collective_id=0),
    )(x, Win0, Wout0, Win1, Wout1, Win2, Wout2)


# device time: 20836 ns/iter; 2.1588x vs baseline; 2.1588x over previous
import jax
import jax.numpy as jnp
from jax import lax
from jax.experimental import pallas as pl
from jax.experimental.pallas import tpu as pltpu

N_DEV = 4

_DIRS = ("R", "R", "L", "L", "R", "R", "L")


def kernel(x, Win0, Wout0, Win1, Wout1, Win2, Wout2):
    m_per, d = x.shape
    h_per = Win0.shape[1]
    M = N_DEV * m_per

    def body(x_ref, win0, wout0, win1, wout1, win2, wout2, out_ref,
             xc, wic, woc, xg, send_sems, recv_sems):
        my = lax.axis_index("i")
        left = lax.rem(my + N_DEV - 1, N_DEV)
        right = lax.rem(my + 1, N_DEV)

        def chunk(t, k):
            if t == 0:
                return xc.at[k]
            l, kind = (t - 1) // 2, (t - 1) % 2
            return wic.at[l, k] if kind == 0 else woc.at[l, k]

        barrier = pltpu.get_barrier_semaphore()
        for nbr in (left, right):
            pl.semaphore_signal(barrier, inc=1, device_id=(nbr,),
                                device_id_type=pl.DeviceIdType.MESH)
        pl.semaphore_wait(barrier, 2)

        xc[0] = x_ref[:, :].astype(jnp.bfloat16)
        for l, (wi, wo) in enumerate(((win0, wout0), (win1, wout1),
                                      (win2, wout2))):
            wic[l, 0] = wi[:, :].astype(jnp.bfloat16)
            woc[l, 0] = wo[:, :].astype(jnp.bfloat16)

        rdmas = []

        for t in range(7):
            for di, (nbr, dst_k) in enumerate(((right, 3), (left, 1))):
                idx = 2 * t + di
                r = pltpu.make_async_remote_copy(
                    src_ref=chunk(t, 0), dst_ref=chunk(t, dst_k),
                    send_sem=send_sems.at[idx], recv_sem=recv_sems.at[idx],
                    device_id=(nbr,), device_id_type=pl.DeviceIdType.MESH)
                r.start()
                rdmas.append(r)

        def wait_recv(t, k, idx):
            pltpu.make_async_remote_copy(
                src_ref=chunk(t, k), dst_ref=chunk(t, k),
                send_sem=send_sems.at[idx], recv_sem=recv_sems.at[idx],
                device_id=(left,),
                device_id_type=pl.DeviceIdType.MESH).wait_recv()

        for t in range(7):
            idx2 = 14 + t
            if _DIRS[t] == "R":
                wait_recv(t, 3, 2 * t)
                src_k, nbr = 3, right
            else:
                wait_recv(t, 1, 2 * t + 1)
                src_k, nbr = 1, left
            r = pltpu.make_async_remote_copy(
                src_ref=chunk(t, src_k), dst_ref=chunk(t, 2),
                send_sem=send_sems.at[idx2], recv_sem=recv_sems.at[idx2],
                device_id=(nbr,), device_id_type=pl.DeviceIdType.MESH)
            r.start()
            rdmas.append(r)

        def wait_rest(t):
            if _DIRS[t] == "R":
                wait_recv(t, 1, 2 * t + 1)
            else:
                wait_recv(t, 3, 2 * t)
            wait_recv(t, 2, 14 + t)

        wait_rest(0)
        xg[pl.ds(my * m_per, m_per), :] = xc[0]
        for k in range(1, N_DEV):
            pos = lax.rem(my + k, N_DEV)
            xg[pl.ds(pos * m_per, m_per), :] = xc[k]
        xcur = xg[:, :]

        for l in range(3):
            wait_rest(1 + 2 * l)
            wait_rest(2 + 2 * l)
            p = jnp.zeros((M, d), jnp.float32)
            for k in range(N_DEV):
                hk = jnp.dot(xcur, wic[l, k],
                             preferred_element_type=jnp.float32)
                hk = jnp.maximum(hk, 0.0).astype(jnp.bfloat16)
                p = p + jnp.dot(hk, woc[l, k],
                                preferred_element_type=jnp.float32)
            if l < 2:
                xcur = p.astype(jnp.bfloat16)
            else:
                out_ref[:, :] = p

        for r in rdmas:
            r.wait_send()

    return pl.pallas_call(
        body,
        out_shape=jax.ShapeDtypeStruct((M, d), jnp.float32),
        in_specs=[pl.BlockSpec(memory_space=pltpu.VMEM)] * 7,
        out_specs=pl.BlockSpec(memory_space=pltpu.VMEM),
        scratch_shapes=[
            pltpu.VMEM((N_DEV, m_per, d), jnp.bfloat16),
            pltpu.VMEM((3, N_DEV, d, h_per), jnp.bfloat16),
            pltpu.VMEM((3, N_DEV, h_per, d), jnp.bfloat16),
            pltpu.VMEM((M, d), jnp.bfloat16),
            pltpu.SemaphoreType.DMA((21,)),
            pltpu.SemaphoreType.DMA((21,)),
        ],
        compiler_params=pltpu.CompilerParams(collective_id=0),
    )(x, Win0, Wout0, Win1, Wout1, Win2, Wout2)
